# baseline (speedup 1.0000x reference)
import functools

import jax
import jax.numpy as jnp
from jax import lax
from jax.experimental import pallas as pl
from jax.experimental.pallas import tpu as pltpu
from jax.experimental.pallas import tpu_sc as plsc

B = 1024
IN_DIM = 4096
OUT_DIM = 4096
FAN_IN = 8
L = 16
NC, NS = 2, 16
NW = NC * NS
RPW = B // NW
G = OUT_DIM // L
C = IN_DIM // L
PB = 4
NPASS = RPW // (2 * PB)


def _tec_body(x_hbm, ip_hbm, wp_hbm, b_hbm, out_hbm,
              x0, x1, x2, x3, y0, y1, y2, y3, y4, y5, y6, y7,
              ipv, wv, bv, sem):
    xp = (x0, x1, x2, x3)
    yr = (y0, y1, y2, y3, y4, y5, y6, y7)
    wid = lax.axis_index("s") * NC + lax.axis_index("c")
    base = wid * RPW
    pltpu.sync_copy(ip_hbm, ipv)
    pltpu.sync_copy(wp_hbm, wv)
    pltpu.sync_copy(b_hbm, bv)

    def pass_body(p, carry):
        row = base + p * 2 * PB
        cps = [pltpu.async_copy(x_hbm.at[row + r], yr[r], sem)
               for r in range(2 * PB)]
        for c in cps:
            c.wait()

        @plsc.parallel_loop(0, C, 1, unroll=4)
        def pack_body(c):
            o = c * L
            for j in range(PB):
                pk = plsc.pack(yr[2 * j][pl.ds(o, L)],
                               yr[2 * j + 1][pl.ds(o, L)],
                               format=plsc.PackFormat.INTERLEAVED)
                xp[j][pl.ds(o, L)] = plsc.bitcast(pk, jnp.int32)

        @plsc.parallel_loop(0, G, 1, unroll=4)
        def grp_body(g):
            o = g * L
            acc = [None] * PB
            for t in range(FAN_IN // 2):
                iw = plsc.bitcast(ipv[t, pl.ds(o, L)], jnp.int16)
                iv0, iv1 = plsc.unpack(iw, format=plsc.PackFormat.INTERLEAVED)
                ww = plsc.bitcast(wv[t, pl.ds(o, L)], jnp.bfloat16)
                wa, wb = plsc.unpack(ww, format=plsc.PackFormat.INTERLEAVED)
                for k, ivec, wf in ((2 * t, iv0, wa), (2 * t + 1, iv1, wb)):
                    wvec = plsc.pack(wf, wf,
                                     format=plsc.PackFormat.INTERLEAVED)
                    for j in range(PB):
                        gb = plsc.bitcast(plsc.load_gather(xp[j], [ivec]),
                                          jnp.bfloat16)
                        t2 = gb * wvec
                        acc[j] = t2 if k == 0 else acc[j] + t2
            bvec = bv[pl.ds(o, L)]
            for j in range(PB):
                lo, hi = plsc.unpack(acc[j],
                                     format=plsc.PackFormat.INTERLEAVED)
                for r, val in ((2 * j, lo), (2 * j + 1, hi)):
                    yr[r][pl.ds(o, L)] = 1.0 / (1.0 + jnp.exp(bvec - val))

        ocps = [pltpu.async_copy(yr[r], out_hbm.at[row + r], sem)
                for r in range(2 * PB)]
        for c in ocps:
            c.wait()
        return carry

    lax.fori_loop(0, NPASS, pass_body, 0)


def kernel(x, idx, w, theta, s_raw):
    s = jax.nn.softplus(s_raw) + 1e-6
    bterm = s * theta

    iu = jnp.asarray(idx, jnp.uint32).T
    ipack = lax.bitcast_convert_type(iu[0::2] | (iu[1::2] << 16), jnp.int32)

    wu = lax.bitcast_convert_type((w * s[:, None]).T.astype(jnp.bfloat16),
                                  jnp.uint16).astype(jnp.uint32)
    wpack = lax.bitcast_convert_type(wu[0::2] | (wu[1::2] << 16), jnp.int32)

    mesh = plsc.VectorSubcoreMesh(core_axis_name="c", subcore_axis_name="s")
    run = functools.partial(
        pl.kernel,
        mesh=mesh,
        compiler_params=pltpu.CompilerParams(needs_layout_passes=False),
        out_type=jax.ShapeDtypeStruct((B, OUT_DIM), jnp.float32),
        scratch_types=(
            [pltpu.VMEM((IN_DIM,), jnp.int32) for _ in range(PB)]
            + [pltpu.VMEM((OUT_DIM,), jnp.float32) for _ in range(2 * PB)]
            + [
                pltpu.VMEM((FAN_IN // 2, OUT_DIM), jnp.int32),
                pltpu.VMEM((FAN_IN // 2, OUT_DIM), jnp.int32),
                pltpu.VMEM((OUT_DIM,), jnp.float32),
                pltpu.SemaphoreType.DMA,
            ]
        ),
    )(_tec_body)
    return run(x, ipack, wpack, bterm)


# speedup vs baseline: 1.0620x; 1.0620x over previous
import functools

import jax
import jax.numpy as jnp
from jax import lax
from jax.experimental import pallas as pl
from jax.experimental.pallas import tpu as pltpu
from jax.experimental.pallas import tpu_sc as plsc

B = 1024
IN_DIM = 4096
OUT_DIM = 4096
FAN_IN = 8
L = 16
NC, NS = 2, 16
NW = NC * NS
RPW = B // NW
G = OUT_DIM // L
C = IN_DIM // L
PB = 4
NPASS = RPW // (2 * PB)


def _tec_body(x_hbm, ip_hbm, wp_hbm, b_hbm, out_hbm,
              x0, x1, x2, x3, y0, y1, y2, y3, y4, y5, y6, y7,
              ipv, wv, bv, sem):
    xp = (x0, x1, x2, x3)
    yr = (y0, y1, y2, y3, y4, y5, y6, y7)
    wid = lax.axis_index("s") * NC + lax.axis_index("c")
    base = wid * RPW
    pltpu.sync_copy(ip_hbm, ipv)
    pltpu.sync_copy(wp_hbm, wv)
    pltpu.sync_copy(b_hbm, bv)

    def pass_body(p, carry):
        row = base + p * 2 * PB
        cps = [pltpu.async_copy(x_hbm.at[row + r], yr[r], sem)
               for r in range(2 * PB)]
        for c in cps:
            c.wait()

        @plsc.parallel_loop(0, C, 1, unroll=4)
        def pack_body(c):
            o = c * L
            for j in range(PB):
                pk = plsc.pack(yr[2 * j][pl.ds(o, L)],
                               yr[2 * j + 1][pl.ds(o, L)],
                               format=plsc.PackFormat.INTERLEAVED)
                xp[j][pl.ds(o, L)] = plsc.bitcast(pk, jnp.int32)

        @plsc.parallel_loop(0, G, 1, unroll=2)
        def grp_body(g):
            o = g * L
            acc = [None] * PB
            for t in range(FAN_IN // 2):
                iw = plsc.bitcast(ipv[t, pl.ds(o, L)], jnp.int16)
                iv0, iv1 = plsc.unpack(iw, format=plsc.PackFormat.INTERLEAVED)
                ww = plsc.bitcast(wv[t, pl.ds(o, L)], jnp.bfloat16)
                wa, wb = plsc.unpack(ww, format=plsc.PackFormat.INTERLEAVED)
                for k, ivec, wf in ((2 * t, iv0, wa), (2 * t + 1, iv1, wb)):
                    wvec = plsc.pack(wf, wf,
                                     format=plsc.PackFormat.INTERLEAVED)
                    for j in range(PB):
                        gb = plsc.bitcast(plsc.load_gather(xp[j], [ivec]),
                                          jnp.bfloat16)
                        t2 = gb * wvec
                        acc[j] = t2 if k == 0 else acc[j] + t2
            bb = plsc.bitcast(bv[pl.ds(o, L)], jnp.bfloat16)
            for j in range(PB):
                ypk = 1.0 / (1.0 + jnp.exp(bb - acc[j]))
                lo, hi = plsc.unpack(ypk,
                                     format=plsc.PackFormat.INTERLEAVED)
                yr[2 * j][pl.ds(o, L)] = lo
                yr[2 * j + 1][pl.ds(o, L)] = hi

        ocps = [pltpu.async_copy(yr[r], out_hbm.at[row + r], sem)
                for r in range(2 * PB)]
        for c in ocps:
            c.wait()
        return carry

    lax.fori_loop(0, NPASS, pass_body, 0)


def kernel(x, idx, w, theta, s_raw):
    s = jax.nn.softplus(s_raw) + 1e-6
    bu = lax.bitcast_convert_type((s * theta).astype(jnp.bfloat16),
                                  jnp.uint16).astype(jnp.uint32)
    bterm = lax.bitcast_convert_type(bu | (bu << 16), jnp.int32)

    iu = jnp.asarray(idx, jnp.uint32).T
    ipack = lax.bitcast_convert_type(iu[0::2] | (iu[1::2] << 16), jnp.int32)

    wu = lax.bitcast_convert_type((w * s[:, None]).T.astype(jnp.bfloat16),
                                  jnp.uint16).astype(jnp.uint32)
    wpack = lax.bitcast_convert_type(wu[0::2] | (wu[1::2] << 16), jnp.int32)

    mesh = plsc.VectorSubcoreMesh(core_axis_name="c", subcore_axis_name="s")
    run = functools.partial(
        pl.kernel,
        mesh=mesh,
        compiler_params=pltpu.CompilerParams(needs_layout_passes=False),
        out_type=jax.ShapeDtypeStruct((B, OUT_DIM), jnp.float32),
        scratch_types=(
            [pltpu.VMEM((IN_DIM,), jnp.int32) for _ in range(PB)]
            + [pltpu.VMEM((OUT_DIM,), jnp.float32) for _ in range(2 * PB)]
            + [
                pltpu.VMEM((FAN_IN // 2, OUT_DIM), jnp.int32),
                pltpu.VMEM((FAN_IN // 2, OUT_DIM), jnp.int32),
                pltpu.VMEM((OUT_DIM,), jnp.int32),
                pltpu.SemaphoreType.DMA,
            ]
        ),
    )(_tec_body)
    return run(x, ipack, wpack, bterm)


# speedup vs baseline: 1.1286x; 1.0627x over previous
import functools

import jax
import jax.numpy as jnp
from jax import lax
from jax.experimental import pallas as pl
from jax.experimental.pallas import tpu as pltpu
from jax.experimental.pallas import tpu_sc as plsc

B = 1024
IN_DIM = 4096
OUT_DIM = 4096
FAN_IN = 8
L = 16
NC, NS = 2, 16
NW = NC * NS
RPW = B // NW
G = OUT_DIM // L
C = IN_DIM // L
PB = 4
RB = 2 * PB
NPASS = RPW // RB


def _tec_body(x_hbm, ip_hbm, wp_hbm, b_hbm, out_hbm,
              x0, x1, x2, x3, y0, y1, y2, y3, y4, y5, y6, y7,
              stg, ipv, wv, bv, sem_i, sem_o):
    xp = (x0, x1, x2, x3)
    yr = (y0, y1, y2, y3, y4, y5, y6, y7)
    wid = lax.axis_index("s") * NC + lax.axis_index("c")
    base = wid * RPW
    for r in range(RB):
        pltpu.async_copy(x_hbm.at[base + r], stg.at[r], sem_i)
    pltpu.sync_copy(ip_hbm, ipv)
    pltpu.sync_copy(wp_hbm, wv)
    pltpu.sync_copy(b_hbm, bv)

    def pass_body(p, carry):
        row = base + p * RB
        for r in range(RB):
            pltpu.make_async_copy(x_hbm.at[base + r], stg.at[r], sem_i).wait()

        @plsc.parallel_loop(0, C, 1, unroll=4)
        def pack_body(c):
            o = c * L
            for j in range(PB):
                pk = plsc.pack(stg[2 * j, pl.ds(o, L)],
                               stg[2 * j + 1, pl.ds(o, L)],
                               format=plsc.PackFormat.INTERLEAVED)
                xp[j][pl.ds(o, L)] = plsc.bitcast(pk, jnp.int32)

        rown = base + jnp.minimum(p + 1, NPASS - 1) * RB
        for r in range(RB):
            pltpu.async_copy(x_hbm.at[rown + r], stg.at[r], sem_i)

        @pl.when(p > 0)
        def _():
            for r in range(RB):
                pltpu.make_async_copy(yr[r], out_hbm.at[base + r],
                                      sem_o).wait()

        @plsc.parallel_loop(0, G, 1, unroll=2)
        def grp_body(g):
            o = g * L
            acc = [None] * PB
            for t in range(FAN_IN // 2):
                iw = plsc.bitcast(ipv[t, pl.ds(o, L)], jnp.int16)
                iv0, iv1 = plsc.unpack(iw, format=plsc.PackFormat.INTERLEAVED)
                ww = plsc.bitcast(wv[t, pl.ds(o, L)], jnp.bfloat16)
                wa, wb = plsc.unpack(ww, format=plsc.PackFormat.INTERLEAVED)
                for k, ivec, wf in ((2 * t, iv0, wa), (2 * t + 1, iv1, wb)):
                    wvec = plsc.pack(wf, wf,
                                     format=plsc.PackFormat.INTERLEAVED)
                    for j in range(PB):
                        gb = plsc.bitcast(plsc.load_gather(xp[j], [ivec]),
                                          jnp.bfloat16)
                        t2 = gb * wvec
                        acc[j] = t2 if k == 0 else acc[j] + t2
            bb = plsc.bitcast(bv[pl.ds(o, L)], jnp.bfloat16)
            for j in range(PB):
                ypk = 1.0 / (1.0 + jnp.exp(bb - acc[j]))
                lo, hi = plsc.unpack(ypk,
                                     format=plsc.PackFormat.INTERLEAVED)
                yr[2 * j][pl.ds(o, L)] = lo
                yr[2 * j + 1][pl.ds(o, L)] = hi

        for r in range(RB):
            pltpu.async_copy(yr[r], out_hbm.at[row + r], sem_o)
        return carry

    lax.fori_loop(0, NPASS, pass_body, 0)
    for r in range(RB):
        pltpu.make_async_copy(yr[r], out_hbm.at[base + r], sem_o).wait()
        pltpu.make_async_copy(x_hbm.at[base + r], stg.at[r], sem_i).wait()


def kernel(x, idx, w, theta, s_raw):
    s = jax.nn.softplus(s_raw) + 1e-6
    bu = lax.bitcast_convert_type((s * theta).astype(jnp.bfloat16),
                                  jnp.uint16).astype(jnp.uint32)
    bterm = lax.bitcast_convert_type(bu | (bu << 16), jnp.int32)

    iu = jnp.asarray(idx, jnp.uint32).T
    ipack = lax.bitcast_convert_type(iu[0::2] | (iu[1::2] << 16), jnp.int32)

    wu = lax.bitcast_convert_type((w * s[:, None]).T.astype(jnp.bfloat16),
                                  jnp.uint16).astype(jnp.uint32)
    wpack = lax.bitcast_convert_type(wu[0::2] | (wu[1::2] << 16), jnp.int32)

    mesh = plsc.VectorSubcoreMesh(core_axis_name="c", subcore_axis_name="s")
    run = functools.partial(
        pl.kernel,
        mesh=mesh,
        compiler_params=pltpu.CompilerParams(needs_layout_passes=False),
        out_type=jax.ShapeDtypeStruct((B, OUT_DIM), jnp.float32),
        scratch_types=(
            [pltpu.VMEM((IN_DIM,), jnp.int32) for _ in range(PB)]
            + [pltpu.VMEM((OUT_DIM,), jnp.float32) for _ in range(RB)]
            + [
                pltpu.VMEM((RB, IN_DIM), jnp.float32),
                pltpu.VMEM((FAN_IN // 2, OUT_DIM), jnp.int32),
                pltpu.VMEM((FAN_IN // 2, OUT_DIM), jnp.int32),
                pltpu.VMEM((OUT_DIM,), jnp.int32),
                pltpu.SemaphoreType.DMA,
                pltpu.SemaphoreType.DMA,
            ]
        ),
    )(_tec_body)
    return run(x, ipack, wpack, bterm)


# speedup vs baseline: 1.1287x; 1.0001x over previous
import functools

import jax
import jax.numpy as jnp
from jax import lax
from jax.experimental import pallas as pl
from jax.experimental.pallas import tpu as pltpu
from jax.experimental.pallas import tpu_sc as plsc

B = 1024
IN_DIM = 4096
OUT_DIM = 4096
FAN_IN = 8
L = 16
NC, NS = 2, 16
NW = NC * NS
RPW = B // NW
G = OUT_DIM // L
C = IN_DIM // L
PB = 4
RB = 2 * PB
NPASS = RPW // RB


def _tec_body(x_hbm, ip_hbm, wp_hbm, b_hbm, out_hbm,
              x0, x1, x2, x3, y0, y1, y2, y3, y4, y5, y6, y7,
              stg, ipv, wv, bv, sem_i, sem_o):
    xp = (x0, x1, x2, x3)
    yr = (y0, y1, y2, y3, y4, y5, y6, y7)
    wid = lax.axis_index("s") * NC + lax.axis_index("c")
    base = wid * RPW
    for r in range(RB):
        pltpu.async_copy(x_hbm.at[base + r], stg.at[r], sem_i)
    pltpu.sync_copy(ip_hbm, ipv)
    pltpu.sync_copy(wp_hbm, wv)
    pltpu.sync_copy(b_hbm, bv)

    def pass_body(p, carry):
        row = base + p * RB
        for r in range(RB):
            pltpu.make_async_copy(x_hbm.at[base + r], stg.at[r], sem_i).wait()

        @plsc.parallel_loop(0, C, 1, unroll=8)
        def pack_body(c):
            o = c * L
            for j in range(PB):
                pk = plsc.pack(stg[2 * j, pl.ds(o, L)],
                               stg[2 * j + 1, pl.ds(o, L)],
                               format=plsc.PackFormat.INTERLEAVED)
                xp[j][pl.ds(o, L)] = plsc.bitcast(pk, jnp.int32)

        rown = base + jnp.minimum(p + 1, NPASS - 1) * RB
        for r in range(RB):
            pltpu.async_copy(x_hbm.at[rown + r], stg.at[r], sem_i)

        @pl.when(p > 0)
        def _():
            for r in range(RB):
                pltpu.make_async_copy(yr[r], out_hbm.at[base + r],
                                      sem_o).wait()

        @plsc.parallel_loop(0, G, 1, unroll=2)
        def grp_body(g):
            o = g * L
            acc = [None] * PB
            for t in range(FAN_IN // 2):
                iw = plsc.bitcast(ipv[t, pl.ds(o, L)], jnp.int16)
                iv0, iv1 = plsc.unpack(iw, format=plsc.PackFormat.INTERLEAVED)
                ww = plsc.bitcast(wv[t, pl.ds(o, L)], jnp.bfloat16)
                wa, wb = plsc.unpack(ww, format=plsc.PackFormat.INTERLEAVED)
                for k, ivec, wf in ((2 * t, iv0, wa), (2 * t + 1, iv1, wb)):
                    wvec = plsc.pack(wf, wf,
                                     format=plsc.PackFormat.INTERLEAVED)
                    for j in range(PB):
                        gb = plsc.bitcast(plsc.load_gather(xp[j], [ivec]),
                                          jnp.bfloat16)
                        t2 = gb * wvec
                        acc[j] = t2 if k == 0 else acc[j] + t2
            bb = plsc.bitcast(bv[pl.ds(o, L)], jnp.bfloat16)
            for j in range(PB):
                ypk = 1.0 / (1.0 + jnp.exp(bb - acc[j]))
                lo, hi = plsc.unpack(ypk,
                                     format=plsc.PackFormat.INTERLEAVED)
                yr[2 * j][pl.ds(o, L)] = lo
                yr[2 * j + 1][pl.ds(o, L)] = hi

        for r in range(RB):
            pltpu.async_copy(yr[r], out_hbm.at[row + r], sem_o)
        return carry

    lax.fori_loop(0, NPASS, pass_body, 0)
    for r in range(RB):
        pltpu.make_async_copy(yr[r], out_hbm.at[base + r], sem_o).wait()
        pltpu.make_async_copy(x_hbm.at[base + r], stg.at[r], sem_i).wait()


def kernel(x, idx, w, theta, s_raw):
    s = jax.nn.softplus(s_raw) + 1e-6
    bu = lax.bitcast_convert_type((s * theta).astype(jnp.bfloat16),
                                  jnp.uint16).astype(jnp.uint32)
    bterm = lax.bitcast_convert_type(bu | (bu << 16), jnp.int32)

    iu = jnp.asarray(idx, jnp.uint32).T
    ipack = lax.bitcast_convert_type(iu[0::2] | (iu[1::2] << 16), jnp.int32)

    wu = lax.bitcast_convert_type((w * s[:, None]).T.astype(jnp.bfloat16),
                                  jnp.uint16).astype(jnp.uint32)
    wpack = lax.bitcast_convert_type(wu[0::2] | (wu[1::2] << 16), jnp.int32)

    mesh = plsc.VectorSubcoreMesh(core_axis_name="c", subcore_axis_name="s")
    run = functools.partial(
        pl.kernel,
        mesh=mesh,
        compiler_params=pltpu.CompilerParams(needs_layout_passes=False),
        out_type=jax.ShapeDtypeStruct((B, OUT_DIM), jnp.float32),
        scratch_types=(
            [pltpu.VMEM((IN_DIM,), jnp.int32) for _ in range(PB)]
            + [pltpu.VMEM((OUT_DIM,), jnp.float32) for _ in range(RB)]
            + [
                pltpu.VMEM((RB, IN_DIM), jnp.float32),
                pltpu.VMEM((FAN_IN // 2, OUT_DIM), jnp.int32),
                pltpu.VMEM((FAN_IN // 2, OUT_DIM), jnp.int32),
                pltpu.VMEM((OUT_DIM,), jnp.int32),
                pltpu.SemaphoreType.DMA,
                pltpu.SemaphoreType.DMA,
            ]
        ),
    )(_tec_body)
    return run(x, ipack, wpack, bterm)
